# baseline (speedup 1.0000x reference)
import jax
import jax.numpy as jnp
from jax import lax
from jax.experimental import pallas as pl
from jax.experimental.pallas import tpu as pltpu

H = 384
W = 384
NUM_KPT = 100
R = 3
NEG = float("-inf")


def _mp7(x):
    colpad = jnp.full((H, R), NEG, dtype=x.dtype)
    a = jnp.concatenate([colpad, x, colpad], axis=1)
    a = jnp.maximum(a[:, :-1], a[:, 1:])
    a = jnp.maximum(a[:, :-2], a[:, 2:])
    h = jnp.maximum(a[:, :-3], a[:, 3:])
    rowpad = jnp.full((R, W), NEG, dtype=x.dtype)
    b = jnp.concatenate([rowpad, h, rowpad], axis=0)
    b = jnp.maximum(b[:-1, :], b[1:, :])
    b = jnp.maximum(b[:-2, :], b[2:, :])
    return jnp.maximum(b[:-3, :], b[3:, :])


def _nms(x):
    mask = x == _mp7(x)
    for _ in range(2):
        suppf = _mp7(mask.astype(jnp.float32))
        supp = suppf > 0
        ss = jnp.where(supp, 0.0, x)
        nm = ss == _mp7(ss)
        mask = mask | (nm & (~supp))
    return jnp.where(mask & (x > 0), x, NEG)


def _body(score_ref, points_ref, feature_any,
          kpts_ref, fea_ref, pix_ref,
          m0_ref, m1_ref, fea_stage_ref, pts_stage_ref, sem_f, sem_p):
    m_refs = (m0_ref, m1_ref)
    rmax2s = []
    for b in range(2):
        m = _nms(score_ref[b, 0])
        m_refs[b][...] = m
        rmax2s.append(jnp.max(m.reshape(8, 48, W), axis=2))
    INF = jnp.float32(float("inf"))

    row2_iota = (lax.broadcasted_iota(jnp.int32, (8, 48), 0) * 48
                 + lax.broadcasted_iota(jnp.int32, (8, 48), 1))
    col_iota = lax.broadcasted_iota(jnp.int32, (1, W), 1)
    k_iota = lax.broadcasted_iota(jnp.int32, (NUM_KPT, 128), 0)
    off_iota = lax.broadcasted_iota(jnp.int32, (NUM_KPT, 128), 1)
    BIG = jnp.int32(1 << 30)

    def step(k, carry):
        rm = [carry[0], carry[1]]
        oh = [carry[2], carry[3]]
        vl = [carry[4], carry[5]]
        cl = [carry[6], carry[7]]
        for b in range(2):
            rmax2 = rm[b]
            v = jnp.max(rmax2, axis=(0, 1), keepdims=True)
            rsel = rmax2 == v
            r = jnp.min(jnp.where(rsel, row2_iota, BIG))
            row = m_refs[b][pl.ds(r, 1), :]
            pickmask = row2_iota == r
            vl11 = jnp.min(jnp.where(pickmask, vl[b], INF), axis=(0, 1),
                           keepdims=True)
            cl11 = jnp.max(jnp.where(pickmask, cl[b], -1), axis=(0, 1),
                           keepdims=True)
            eligible = (row == v[0:1, 0:1]) & (
                (vl11 != v[0:1, 0:1]) | (col_iota > cl11))
            c = jnp.min(jnp.where(eligible, col_iota, BIG))

            consumed = (row > v[0:1, 0:1]) | (
                (row == v[0:1, 0:1]) & (col_iota <= c))
            rem = jnp.where(consumed, NEG, row)
            rowmax = jnp.max(rem, axis=(0, 1), keepdims=True)
            rm[b] = jnp.where(pickmask, rowmax[0:1, 0:1], rmax2)
            vl[b] = jnp.where(pickmask, v[0:1, 0:1], vl[b])
            cl[b] = jnp.where(pickmask, c, cl[b])

            pix_ref[b, k, 0] = r
            pix_ref[b, k, 1] = c

            c128 = pl.multiple_of((c // 128) * 128, 128)
            oh[b] = oh[b] + jnp.where(
                (k_iota == k) & (off_iota == c - c128), 1.0, 0.0)
            pltpu.make_async_copy(
                feature_any.at[b, :, r, pl.ds(c128, 128)],
                fea_stage_ref.at[b, k],
                sem_f,
            ).start()
            pltpu.make_async_copy(
                points_ref.at[b, :, r, pl.ds(c128, 128)],
                pts_stage_ref.at[b, :, k],
                sem_p,
            ).start()
        return (rm[0], rm[1], oh[0], oh[1],
                vl[0], vl[1], cl[0], cl[1])

    zoh = jnp.zeros((NUM_KPT, 128), jnp.float32)
    zvl = jnp.full((8, 48), float("inf"), jnp.float32)
    zcl = jnp.full((8, 48), -1, jnp.int32)
    _, _, oh0, oh1, _, _, _, _ = lax.fori_loop(
        0, NUM_KPT, step,
        (rmax2s[0], rmax2s[1], zoh, zoh, zvl, zvl, zcl, zcl), unroll=2)

    for b in range(2):
        pltpu.make_async_copy(
            feature_any.at[b, pl.ds(0, NUM_KPT), pl.ds(0, 128),
                           pl.ds(0, 128)],
            fea_stage_ref.at[b],
            sem_f,
        ).wait()
        pltpu.make_async_copy(
            feature_any.at[:, pl.ds(0, NUM_KPT), 0, pl.ds(0, 128)],
            pts_stage_ref.at[b],
            sem_p,
        ).wait()

    for b, onehot in ((0, oh0), (1, oh1)):
        sel_f = jnp.sum(fea_stage_ref[b] * onehot[:, None, :], axis=2)
        fea_ref[b] = sel_f.T
        sel_p = jnp.sum(pts_stage_ref[b] * onehot[None, :, :], axis=2)
        kpts_ref[b] = jnp.concatenate(
            [sel_p.T,
             jnp.zeros((NUM_KPT, 1), jnp.float32),
             jnp.ones((NUM_KPT, 1), jnp.float32)], axis=1)


@jax.jit
def kernel(score_bev, points, feature_bev):
    bsz = score_bev.shape[0]
    kpts, feas, pix = pl.pallas_call(
        _body,
        grid=(1,),
        in_specs=[
            pl.BlockSpec((bsz, 1, H, W), lambda i: (0, 0, 0, 0)),
            pl.BlockSpec((bsz, 2, H, W), lambda i: (0, 0, 0, 0)),
            pl.BlockSpec(memory_space=pl.ANY),
        ],
        out_specs=[
            pl.BlockSpec((bsz, NUM_KPT, 4), lambda i: (0, 0, 0)),
            pl.BlockSpec((bsz, 128, NUM_KPT), lambda i: (0, 0, 0)),
            pl.BlockSpec((bsz, NUM_KPT, 2), lambda i: (0, 0, 0),
                         memory_space=pltpu.SMEM),
        ],
        out_shape=[
            jax.ShapeDtypeStruct((bsz, NUM_KPT, 4), jnp.float32),
            jax.ShapeDtypeStruct((bsz, 128, NUM_KPT), jnp.float32),
            jax.ShapeDtypeStruct((bsz, NUM_KPT, 2), jnp.int32),
        ],
        scratch_shapes=[
            pltpu.VMEM((H, W), jnp.float32),
            pltpu.VMEM((H, W), jnp.float32),
            pltpu.VMEM((2, NUM_KPT, 128, 128), jnp.float32),
            pltpu.VMEM((2, 2, NUM_KPT, 128), jnp.float32),
            pltpu.SemaphoreType.DMA,
            pltpu.SemaphoreType.DMA,
        ],
    )(score_bev, points, feature_bev)
    scores = score_bev.reshape(bsz, H, W)
    return kpts, feas, pix, scores


# speedup vs baseline: 1.2786x; 1.2786x over previous
import jax
import jax.numpy as jnp
from jax import lax
from jax.experimental import pallas as pl
from jax.experimental.pallas import tpu as pltpu

H = 384
W = 384
NUM_KPT = 100
R = 3
NEG = float("-inf")


def _mp7(x):
    colpad = jnp.full((H, R), NEG, dtype=x.dtype)
    a = jnp.concatenate([colpad, x, colpad], axis=1)
    a = jnp.maximum(a[:, :-1], a[:, 1:])
    a = jnp.maximum(a[:, :-2], a[:, 2:])
    h = jnp.maximum(a[:, :-3], a[:, 3:])
    rowpad = jnp.full((R, W), NEG, dtype=x.dtype)
    b = jnp.concatenate([rowpad, h, rowpad], axis=0)
    b = jnp.maximum(b[:-1, :], b[1:, :])
    b = jnp.maximum(b[:-2, :], b[2:, :])
    return jnp.maximum(b[:-3, :], b[3:, :])


def _nms(x):
    mask = x == _mp7(x)
    for _ in range(2):
        suppf = _mp7(mask.astype(jnp.float32))
        supp = suppf > 0
        ss = jnp.where(supp, 0.0, x)
        nm = ss == _mp7(ss)
        mask = mask | (nm & (~supp))
    return jnp.where(mask & (x > 0), x, NEG)


def _body(score_ref, points_ref, feature_any,
          kpts_ref, fea_ref, pix_ref,
          m0_ref, m1_ref, fea_stage_ref, pts_stage_ref, sem_f, sem_p):
    m_refs = (m0_ref, m1_ref)
    rmax2s = []
    for b in range(2):
        m = _nms(score_ref[b, 0])
        m_refs[b][...] = m
        rmax2s.append(jnp.max(m.reshape(8, 48, W), axis=2))

    row2_iota = (lax.broadcasted_iota(jnp.int32, (8, 48), 0) * 48
                 + lax.broadcasted_iota(jnp.int32, (8, 48), 1))
    col_iota = lax.broadcasted_iota(jnp.int32, (1, W), 1)
    k_iota = lax.broadcasted_iota(jnp.int32, (NUM_KPT, 128), 0)
    off_iota = lax.broadcasted_iota(jnp.int32, (NUM_KPT, 128), 1)
    BIG = jnp.int32(1 << 30)

    def step(k, carry):
        rm = [carry[0], carry[1]]
        oh = [carry[2], carry[3]]
        for b in range(2):
            rmax2 = rm[b]
            v = jnp.max(rmax2, axis=(0, 1), keepdims=True)
            r = jnp.min(jnp.where(rmax2 == v, row2_iota, BIG))
            row = m_refs[b][pl.ds(r, 1), :]
            c = jnp.min(jnp.where(row == v[0:1, 0:1], col_iota, BIG))

            new_row = jnp.where(col_iota == c, NEG, row)
            m_refs[b][pl.ds(r, 1), :] = new_row
            rowmax = jnp.max(new_row, axis=(0, 1), keepdims=True)
            rm[b] = jnp.where(row2_iota == r, rowmax[0:1, 0:1], rmax2)

            pix_ref[b, k, 0] = r
            pix_ref[b, k, 1] = c

            c128 = pl.multiple_of((c // 128) * 128, 128)
            oh[b] = oh[b] + jnp.where(
                (k_iota == k) & (off_iota == c - c128), 1.0, 0.0)
            pltpu.make_async_copy(
                feature_any.at[b, :, r, pl.ds(c128, 128)],
                fea_stage_ref.at[b, k],
                sem_f,
            ).start()
            pltpu.make_async_copy(
                points_ref.at[b, :, r, pl.ds(c128, 128)],
                pts_stage_ref.at[b, :, k],
                sem_p,
            ).start()
        return rm[0], rm[1], oh[0], oh[1]

    zoh = jnp.zeros((NUM_KPT, 128), jnp.float32)
    _, _, oh0, oh1 = lax.fori_loop(
        0, NUM_KPT, step, (rmax2s[0], rmax2s[1], zoh, zoh), unroll=2)

    for b in range(2):
        pltpu.make_async_copy(
            feature_any.at[b, pl.ds(0, NUM_KPT), pl.ds(0, 128),
                           pl.ds(0, 128)],
            fea_stage_ref.at[b],
            sem_f,
        ).wait()
        pltpu.make_async_copy(
            feature_any.at[:, pl.ds(0, NUM_KPT), 0, pl.ds(0, 128)],
            pts_stage_ref.at[b],
            sem_p,
        ).wait()

    for b, onehot in ((0, oh0), (1, oh1)):
        sel_f = jnp.sum(fea_stage_ref[b] * onehot[:, None, :], axis=2)
        fea_ref[b] = sel_f.T
        sel_p = jnp.sum(pts_stage_ref[b] * onehot[None, :, :], axis=2)
        kpts_ref[b] = jnp.concatenate(
            [sel_p.T,
             jnp.zeros((NUM_KPT, 1), jnp.float32),
             jnp.ones((NUM_KPT, 1), jnp.float32)], axis=1)


@jax.jit
def kernel(score_bev, points, feature_bev):
    bsz = score_bev.shape[0]
    kpts, feas, pix = pl.pallas_call(
        _body,
        grid=(1,),
        in_specs=[
            pl.BlockSpec((bsz, 1, H, W), lambda i: (0, 0, 0, 0)),
            pl.BlockSpec((bsz, 2, H, W), lambda i: (0, 0, 0, 0)),
            pl.BlockSpec(memory_space=pl.ANY),
        ],
        out_specs=[
            pl.BlockSpec((bsz, NUM_KPT, 4), lambda i: (0, 0, 0)),
            pl.BlockSpec((bsz, 128, NUM_KPT), lambda i: (0, 0, 0)),
            pl.BlockSpec((bsz, NUM_KPT, 2), lambda i: (0, 0, 0),
                         memory_space=pltpu.SMEM),
        ],
        out_shape=[
            jax.ShapeDtypeStruct((bsz, NUM_KPT, 4), jnp.float32),
            jax.ShapeDtypeStruct((bsz, 128, NUM_KPT), jnp.float32),
            jax.ShapeDtypeStruct((bsz, NUM_KPT, 2), jnp.int32),
        ],
        scratch_shapes=[
            pltpu.VMEM((H, W), jnp.float32),
            pltpu.VMEM((H, W), jnp.float32),
            pltpu.VMEM((2, NUM_KPT, 128, 128), jnp.float32),
            pltpu.VMEM((2, 2, NUM_KPT, 128), jnp.float32),
            pltpu.SemaphoreType.DMA,
            pltpu.SemaphoreType.DMA,
        ],
    )(score_bev, points, feature_bev)
    scores = score_bev.reshape(bsz, H, W)
    return kpts, feas, pix, scores
